# Initial kernel scaffold; baseline (speedup 1.0000x reference)
import jax
import jax.numpy as jnp
from jax.experimental import pallas as pl


def kernel(inputs, token_table, row_table, col_table):
    raise NotImplementedError("write your pallas kernel here")


# speedup vs baseline: 2.5032x; 2.5032x over previous
import functools

import jax
import jax.numpy as jnp
from jax import lax
from jax.experimental import pallas as pl
from jax.experimental.pallas import tpu as pltpu
from jax.experimental.pallas import tpu_sc as plsc

VOCAB = 100000
D = 128
GRID = 30
B = 1024
P = GRID * GRID
HALF = P // 2
CHUNK = 90
NCHUNK = HALF // CHUNK
PAIR = 2
NPAIR = 32
LANES = 16
NVREG = D // LANES


def _pos_tc_body(row_ref, col_ref, out_ref):
    col = col_ref[...]
    for i in range(GRID):
        out_ref[pl.ds(GRID * i, GRID), :] = row_ref[i, :][None, :] + col


def _build_pos(row_table, col_table):
    return pl.pallas_call(
        _pos_tc_body,
        out_shape=jax.ShapeDtypeStruct((P, D), jnp.float32),
    )(row_table, col_table)


def _sc_body(idx_hbm, table_hbm, pos_hbm, out_hbm, idx_v, pos_v, rows_v, sem):
    c = lax.axis_index("c")
    s = lax.axis_index("s")

    pltpu.sync_copy(pos_hbm.at[c], pos_v)

    def pair_body(i, carry):
        bp = s * NPAIR + i
        pltpu.sync_copy(idx_hbm.at[bp], idx_v)
        for bi in range(PAIR):
            b = bp * PAIR + bi
            for j in range(NCHUNK):
                idx_row = idx_v.at[bi * (2 * NCHUNK) + c * NCHUNK + j]
                pltpu.async_copy(table_hbm.at[idx_row], rows_v, sem).wait()

                def row_body(r, _):
                    poff = j * CHUNK + r
                    for v in range(NVREG):
                        sl = pl.ds(v * LANES, LANES)
                        plsc.addupdate(rows_v.at[r, sl], pos_v[poff, sl])
                    return 0

                lax.fori_loop(0, CHUNK, row_body, 0)

                cid = b * (P // CHUNK) + c * NCHUNK + j
                pltpu.sync_copy(rows_v, out_hbm.at[cid])
        return carry

    lax.fori_loop(0, NPAIR, pair_body, 0)


def _sc_gather(idx3, token_table, pos):
    mesh = plsc.VectorSubcoreMesh(core_axis_name="c", subcore_axis_name="s")
    run = pl.kernel(
        _sc_body,
        out_type=jax.ShapeDtypeStruct((B * P // CHUNK, CHUNK, D), jnp.float32),
        mesh=mesh,
        scratch_types=[
            pltpu.VMEM((PAIR * 2 * NCHUNK, CHUNK), jnp.int32),
            pltpu.VMEM((HALF, D), jnp.float32),
            pltpu.VMEM((CHUNK, D), jnp.float32),
            pltpu.SemaphoreType.DMA,
        ],
    )
    return run(idx3, token_table, pos)


@jax.jit
def kernel(inputs, token_table, row_table, col_table):
    pos = _build_pos(row_table, col_table).reshape(2, HALF, D)
    idx3 = inputs.astype(jnp.int32).reshape(B // PAIR, PAIR * 2 * NCHUNK, CHUNK)
    out = _sc_gather(idx3, token_table, pos)
    return out.reshape(B, P, D)


# speedup vs baseline: 3.3408x; 1.3346x over previous
import functools

import jax
import jax.numpy as jnp
from jax import lax
from jax.experimental import pallas as pl
from jax.experimental.pallas import tpu as pltpu
from jax.experimental.pallas import tpu_sc as plsc

VOCAB = 100000
D = 128
GRID = 30
B = 1024
P = GRID * GRID
HALF = P // 2
CHUNK = 90
NCHUNK = HALF // CHUNK
PAIR = 2
NPAIR = 32
KPP = PAIR * NCHUNK
NBUF = 4
LOOKAHEAD = 2
LANES = 16
NVREG = D // LANES


def _pos_tc_body(row_ref, col_ref, out_ref):
    col = col_ref[...]
    for i in range(GRID):
        out_ref[pl.ds(GRID * i, GRID), :] = row_ref[i, :][None, :] + col


def _build_pos(row_table, col_table):
    return pl.pallas_call(
        _pos_tc_body,
        out_shape=jax.ShapeDtypeStruct((P, D), jnp.float32),
    )(row_table, col_table)


def _sc_body(idx_hbm, table_hbm, pos_hbm, out_hbm,
             idx_v, pos_v, rows_v, sems_g, sems_w):
    c = lax.axis_index("c")
    s = lax.axis_index("s")

    pltpu.sync_copy(pos_hbm.at[c], pos_v)

    def idx_row(k):
        bi, j = divmod(k, NCHUNK)
        return idx_v.at[bi * (2 * NCHUNK) + c * NCHUNK + j]

    def add_pos(k):
        buf = k % NBUF
        j = k % NCHUNK

        def row_body(r2, _):
            for rr in range(2):
                r = r2 * 2 + rr
                poff = j * CHUNK + r
                for v in range(NVREG):
                    sl = pl.ds(v * LANES, LANES)
                    plsc.addupdate(rows_v.at[buf, r, sl], pos_v[poff, sl])
            return 0

        lax.fori_loop(0, CHUNK // 2, row_body, 0)

    def pair_body(p, carry):
        pltpu.sync_copy(idx_hbm.at[s * NPAIR + p], idx_v)

        gathers = {}
        writes = {}

        def start_gather(k):
            gathers[k] = pltpu.async_copy(
                table_hbm.at[idx_row(k)], rows_v.at[k % NBUF],
                sems_g[k % NBUF])

        for k in range(LOOKAHEAD):
            start_gather(k)
        for k in range(KPP):
            gathers.pop(k).wait()
            add_pos(k)
            bi, j = divmod(k, NCHUNK)
            bp = s * NPAIR + p
            cid = (bp * PAIR + bi) * (P // CHUNK) + c * NCHUNK + j
            writes[k] = pltpu.async_copy(rows_v.at[k % NBUF],
                                         out_hbm.at[cid], sems_w[k % NBUF])
            if k + LOOKAHEAD < KPP:
                prev = k + LOOKAHEAD - NBUF
                if prev >= 0:
                    writes.pop(prev).wait()
                start_gather(k + LOOKAHEAD)
        for k in sorted(writes):
            writes[k].wait()
        return carry

    lax.fori_loop(0, NPAIR, pair_body, 0)


def _sc_gather(idx3, token_table, pos):
    mesh = plsc.VectorSubcoreMesh(core_axis_name="c", subcore_axis_name="s")
    run = pl.kernel(
        lambda *refs: _sc_body(refs[0], refs[1], refs[2], refs[3],
                               refs[4], refs[5], refs[6],
                               list(refs[7:7 + NBUF]),
                               list(refs[7 + NBUF:7 + 2 * NBUF])),
        out_type=jax.ShapeDtypeStruct((B * P // CHUNK, CHUNK, D), jnp.float32),
        mesh=mesh,
        scratch_types=[
            pltpu.VMEM((PAIR * 2 * NCHUNK, CHUNK), jnp.int32),
            pltpu.VMEM((HALF, D), jnp.float32),
            pltpu.VMEM((NBUF, CHUNK, D), jnp.float32),
        ] + [pltpu.SemaphoreType.DMA] * (2 * NBUF),
    )
    return run(idx3, token_table, pos)


@jax.jit
def kernel(inputs, token_table, row_table, col_table):
    pos = _build_pos(row_table, col_table).reshape(2, HALF, D)
    idx3 = inputs.astype(jnp.int32).reshape(B // PAIR, PAIR * 2 * NCHUNK, CHUNK)
    out = _sc_gather(idx3, token_table, pos)
    return out.reshape(B, P, D)


# speedup vs baseline: 5.1813x; 1.5509x over previous
import functools

import jax
import jax.numpy as jnp
from jax import lax
from jax.experimental import pallas as pl
from jax.experimental.pallas import tpu as pltpu
from jax.experimental.pallas import tpu_sc as plsc

VOCAB = 100000
D = 128
GRID = 30
B = 1024
P = GRID * GRID
PPAD = 904
HBASE = 448
HPAD = 456
TB = P - 2 * HBASE
SZ = (120, 120, 120, 88)
OFF = (0, 120, 240, 360)
NCH = len(SZ)
BPI = 4
NITER = 16
NBUF = 4
LOOKAHEAD = 2
LANES = 16
NVREG = D // LANES


def _pos_tc_body(row_ref, col_ref, out_ref):
    col = col_ref[...]
    for i in range(GRID):
        out_ref[pl.ds(GRID * i, GRID), :] = row_ref[i, :][None, :] + col
    out_ref[pl.ds(P, PPAD - P), :] = col[: PPAD - P, :]


def _build_pos(row_table, col_table):
    return pl.pallas_call(
        _pos_tc_body,
        out_shape=jax.ShapeDtypeStruct((PPAD, D), jnp.float32),
    )(row_table, col_table)


def _sc_body(idx_hbm, table_hbm, pos_hbm, out_hbm, tail_hbm,
             idx_v, pos_v, rows_v, tail_v, sems_g, sems_w, sem_t):
    c = lax.axis_index("c")
    s = lax.axis_index("s")

    hbase = pl.multiple_of(c * HBASE, 8)

    pltpu.sync_copy(pos_hbm.at[pl.ds(hbase, HPAD)], pos_v)

    def add_pos(k):
        buf, j = k % NBUF, k % NCH

        def row_body(r2, _):
            for rr in range(2):
                r = r2 * 2 + rr
                poff = OFF[j] + r
                for v in range(NVREG):
                    sl = pl.ds(v * LANES, LANES)
                    plsc.addupdate(rows_v.at[buf, r, sl], pos_v[poff, sl])
            return 0

        lax.fori_loop(0, SZ[j] // 2, row_body, 0)

    def iter_body(it, carry):
        b0 = s * (BPI * NITER) + it * BPI
        pltpu.sync_copy(idx_hbm.at[pl.ds(b0 * PPAD, BPI * PPAD)], idx_v)

        gathers = {}
        writes = {}

        def start_gather(k):
            bi, j = divmod(k, NCH)
            ioff = pl.multiple_of(bi * PPAD + hbase + OFF[j], 8)
            gathers[k] = pltpu.async_copy(
                table_hbm.at[idx_v.at[pl.ds(ioff, SZ[j])]],
                rows_v.at[k % NBUF, pl.ds(0, SZ[j])], sems_g[k % NBUF])

        tail_gathers = []

        @pl.when(c == 1)
        def _():
            for bi in range(BPI):
                tail_gathers.append(pltpu.async_copy(
                    table_hbm.at[idx_v.at[pl.ds(bi * PPAD + 2 * HBASE, TB)]],
                    tail_v.at[pl.ds(bi * TB, TB)], sem_t))

        for k in range(LOOKAHEAD):
            start_gather(k)
        for k in range(BPI * NCH):
            bi, j = divmod(k, NCH)
            gathers.pop(k).wait()
            add_pos(k)
            writes[k] = pltpu.async_copy(
                rows_v.at[k % NBUF, pl.ds(0, SZ[j])],
                out_hbm.at[b0 + bi, pl.ds(hbase + OFF[j], SZ[j])],
                sems_w[k % NBUF])
            if k + LOOKAHEAD < BPI * NCH:
                if k - LOOKAHEAD >= 0:
                    writes.pop(k - LOOKAHEAD).wait()
                start_gather(k + LOOKAHEAD)

        @pl.when(c == 1)
        def _():
            for g in tail_gathers:
                g.wait()
            for bi in range(BPI):
                for r in range(TB):
                    for v in range(NVREG):
                        sl = pl.ds(v * LANES, LANES)
                        plsc.addupdate(tail_v.at[bi * TB + r, sl],
                                       pos_v[HBASE + r, sl])
            toff = pl.multiple_of((s * NITER + it) * BPI * TB, 8)
            pltpu.sync_copy(tail_v, tail_hbm.at[pl.ds(toff, BPI * TB)])

        for k in sorted(writes):
            writes[k].wait()
        return carry

    lax.fori_loop(0, NITER, iter_body, 0)


def _sc_gather(idx1, token_table, pos):
    mesh = plsc.VectorSubcoreMesh(core_axis_name="c", subcore_axis_name="s")
    run = pl.kernel(
        lambda *refs: _sc_body(refs[0], refs[1], refs[2], refs[3], refs[4],
                               refs[5], refs[6], refs[7], refs[8],
                               list(refs[9:9 + NBUF]),
                               list(refs[9 + NBUF:9 + 2 * NBUF]),
                               refs[9 + 2 * NBUF]),
        out_type=(jax.ShapeDtypeStruct((B, P, D), jnp.float32),
                  jax.ShapeDtypeStruct((B * TB, D), jnp.float32)),
        mesh=mesh,
        scratch_types=[
            pltpu.VMEM((BPI * PPAD,), jnp.int32),
            pltpu.VMEM((HPAD, D), jnp.float32),
            pltpu.VMEM((NBUF, SZ[0], D), jnp.float32),
            pltpu.VMEM((BPI * TB, D), jnp.float32),
        ] + [pltpu.SemaphoreType.DMA] * (2 * NBUF + 1),
    )
    return run(idx1, token_table, pos)


@jax.jit
def kernel(inputs, token_table, row_table, col_table):
    pos = _build_pos(row_table, col_table)
    idx1 = jnp.pad(inputs.astype(jnp.int32), ((0, 0), (0, PPAD - P)))
    out, tail = _sc_gather(idx1.reshape(B * PPAD), token_table, pos)
    tail = tail.reshape(B, TB, D)
    return lax.dynamic_update_slice(out, tail, (0, P - TB, 0))
